# baseline (speedup 1.0000x reference)
import functools

import jax
import jax.numpy as jnp
from jax import lax
from jax.experimental import pallas as pl
from jax.experimental.pallas import tpu as pltpu
from jax.experimental.pallas import tpu_sc as plsc

VOCAB = 1000000
DIM = 64
NC = 2
NS = 16
NW = NC * NS
LANES = 16

CHUNK = 512
SUB = 128
NSUB = CHUNK // SUB


def _make_sc_gather(batch):
    assert batch % (8 * NW) == 0
    per_w = batch // NW
    assert per_w % CHUNK == 0
    nch = per_w // CHUNK

    mesh = plsc.VectorSubcoreMesh(core_axis_name="c", subcore_axis_name="s")

    @functools.partial(
        pl.kernel,
        mesh=mesh,
        compiler_params=pltpu.CompilerParams(
            use_tc_tiling_on_sc=False, needs_layout_passes=False),
        out_type=jax.ShapeDtypeStruct((batch, DIM), jnp.float32),
        scratch_types=[
            pltpu.VMEM((per_w,), jnp.int32),
            pltpu.VMEM((2, CHUNK, DIM), jnp.float32),
            pltpu.VMEM((CHUNK + LANES,), jnp.int32),
            pltpu.SemaphoreType.DMA,
            pltpu.SemaphoreType.DMA,
            pltpu.SemaphoreType.DMA,
        ],
    )
    def grab(tab_hbm, idx_hbm, out_hbm, idx_v, rows_v, pos_v, gsem, osem0, osem1):
        wid = lax.axis_index("s") * NC + lax.axis_index("c")
        base = wid * per_w

        pltpu.make_async_copy(idx_hbm.at[pl.ds(base, per_w)], idx_v, gsem).start()
        pltpu.make_async_copy(idx_hbm.at[pl.ds(base, per_w)], idx_v, gsem).wait()

        def fire_gathers(g, buf):
            off = g * CHUNK
            for s in range(NSUB):
                pltpu.make_async_copy(
                    tab_hbm.at[idx_v.at[pl.ds(off + s * SUB, SUB)]],
                    rows_v.at[buf, pl.ds(s * SUB, SUB), :],
                    gsem,
                ).start()

        def drain_gathers(buf):
            for s in range(NSUB):
                pltpu.make_async_copy(
                    tab_hbm.at[idx_v.at[pl.ds(s * SUB, SUB)]],
                    rows_v.at[buf, pl.ds(s * SUB, SUB), :],
                    gsem,
                ).wait()

        def out_copy(g, buf):
            sem = osem0 if buf == 0 else osem1
            return pltpu.make_async_copy(
                rows_v.at[buf],
                out_hbm.at[pl.ds(base + g * CHUNK, CHUNK)],
                sem,
            )

        fire_gathers(0, 0)

        zeros16 = jnp.zeros((LANES,), jnp.float32)
        iota16 = lax.iota(jnp.int32, LANES)

        def chunk_body(g, carry):
            buf = lax.rem(g, 2)
            drain_gathers(buf)

            @pl.when(g + 1 < nch)
            def _():
                @pl.when(g >= 1)
                def _():
                    @pl.when(buf == 0)
                    def _():
                        out_copy(g - 1, 1).wait()

                    @pl.when(buf == 1)
                    def _():
                        out_copy(g - 1, 0).wait()

                @pl.when(buf == 0)
                def _():
                    fire_gathers(g + 1, 1)

                @pl.when(buf == 1)
                def _():
                    fire_gathers(g + 1, 0)

            off = g * CHUNK

            def scan_step(j, cnt):
                v = idx_v[pl.ds(off + j * LANES, LANES)]
                m = v == 0
                ids = iota16 + j * LANES
                s = m.astype(jnp.int32)
                incl = plsc.cumsum(s)
                plsc.store_scatter(pos_v, [cnt + incl - s], ids, mask=m)
                return cnt + incl[LANES - 1]

            cnt = lax.fori_loop(0, CHUNK // LANES, scan_step, jnp.int32(0))

            def fix_step(i, fcarry):
                p = pos_v[pl.ds(i, LANES)][0]
                for c in range(DIM // LANES):
                    rows_v[buf, p, pl.ds(c * LANES, LANES)] = zeros16
                return fcarry

            lax.fori_loop(0, cnt, fix_step, 0)

            @pl.when(buf == 0)
            def _():
                out_copy(g, 0).start()

            @pl.when(buf == 1)
            def _():
                out_copy(g, 1).start()

            return carry

        lax.fori_loop(0, nch, chunk_body, 0)

        last = nch - 1
        out_copy(last - 1, (last - 1) % 2).wait()
        out_copy(last, last % 2).wait()

    return grab


def kernel(inputs, embeddings):
    batch = inputs.shape[0] * inputs.shape[1]
    idx_flat = inputs.reshape(batch)
    out = _make_sc_gather(batch)(embeddings, idx_flat)
    return out.reshape(inputs.shape[0], inputs.shape[1], DIM)
